# baseline (speedup 1.0000x reference)
import functools

import jax
import jax.numpy as jnp
from jax import lax
from jax.experimental import pallas as pl
from jax.experimental.pallas import tpu as pltpu
from jax.experimental.pallas import tpu_sc as plsc

N = 10000
E = 160000
D_IN = 128
D_H = 32
EC = 16000
KBLK = 16000
EPT = E // 32

_SHIFT = 14
_NCH = E // EC
_U = 8
assert E % EC == 0 and EC % (16 * _U) == 0 and N % 16 == 0
assert E % 32 == 0 and EPT % 8 == 0

_MESH = plsc.VectorSubcoreMesh(core_axis_name="c", subcore_axis_name="s")
_SC_PARAMS = pltpu.CompilerParams(needs_layout_passes=False)


def _k1_body(x_ref, wg_ref, e_ref, o_ref, p_ref):
    o_ref[...] = lax.dot_general(
        wg_ref[...], x_ref[...],
        dimension_numbers=(((0,), (1,)), ((), ())),
        preferred_element_type=jnp.float32)
    p_ref[...] = (e_ref[0:1, :] << _SHIFT) | e_ref[1:2, :]


def _k1(x, Wg, edge_index):
    return pl.pallas_call(
        _k1_body,
        out_shape=(
            jax.ShapeDtypeStruct((D_H, N), jnp.float32),
            jax.ShapeDtypeStruct((1, E), jnp.int32),
        ),
    )(x, Wg, edge_index)


@functools.partial(
    pl.kernel,
    mesh=_MESH,
    compiler_params=_SC_PARAMS,
    out_type=jax.ShapeDtypeStruct((32, N), jnp.float32),
    scratch_types=[
        pltpu.VMEM((EPT,), jnp.int32),
        pltpu.VMEM((N,), jnp.float32),
        pltpu.SemaphoreType.DMA,
    ],
)
def _sc_deg(ep_hbm, out_hbm, ep_v, deg_v, sem_e):
    wid = lax.axis_index("s") * 2 + lax.axis_index("c")

    h = pltpu.async_copy(ep_hbm.at[pl.ds(wid * EPT, EPT)], ep_v, sem_e)

    zero16 = jnp.zeros((16,), jnp.float32)

    def _init(i, c):
        deg_v[pl.ds(i * 16, 16)] = zero16
        return c
    lax.fori_loop(0, N // 16, _init, 0)
    h.wait()

    one16 = jnp.full((16,), 1.0, jnp.float32)
    dmask = jnp.full((16,), (1 << _SHIFT) - 1, jnp.int32)
    nfull = EPT // 16
    rem = EPT - nfull * 16

    def _body(i, c):
        e16 = ep_v[pl.ds(i * 16, 16)]
        plsc.addupdate_scatter(deg_v, [e16 & dmask], one16)
        return c
    lax.fori_loop(0, nfull, _body, 0)

    if rem:
        e16 = ep_v[pl.ds(EPT - 16, 16)]
        mask = lax.iota(jnp.int32, 16) >= (16 - rem)
        plsc.addupdate_scatter(deg_v, [e16 & dmask], one16, mask=mask)

    pltpu.sync_copy(deg_v, out_hbm.at[wid])


def _kd_body(degp_ref, xwt_ref, dinv_ref, gall_ref):
    deg = 1.0 + jnp.sum(degp_ref[...], axis=0, keepdims=True)
    dinv = lax.rsqrt(deg)
    dinv_ref[...] = dinv
    gall_ref[...] = dinv * xwt_ref[...]


def _kd(degP, xwT):
    return pl.pallas_call(
        _kd_body,
        out_shape=(
            jax.ShapeDtypeStruct((1, N), jnp.float32),
            jax.ShapeDtypeStruct((D_H, N), jnp.float32),
        ),
    )(degP, xwT)


_EH = E // 2
_NCHH = _EH // EC
assert _EH % EC == 0


@functools.partial(
    pl.kernel,
    mesh=_MESH,
    compiler_params=_SC_PARAMS,
    out_type=jax.ShapeDtypeStruct((32, 2 * N), jnp.float32),
    scratch_types=[
        pltpu.VMEM((EC,), jnp.int32),
        pltpu.VMEM((EC,), jnp.int32),
        pltpu.VMEM((2 * N,), jnp.float32),
        pltpu.VMEM((2 * N,), jnp.float32),
        pltpu.SemaphoreType.DMA,
        pltpu.SemaphoreType.DMA,
        pltpu.SemaphoreType.DMA,
    ],
)
def _sc_msg(ep_hbm, gall2_hbm, out_hbm, ep_v0, ep_v1, g_v, acc_v,
            sem_0, sem_1, sem_g):
    wid = lax.axis_index("s") * 2 + lax.axis_index("c")
    half = wid // 16
    pair = wid % 16
    ep_b = (ep_v0, ep_v1)
    sem = (sem_0, sem_1)
    dmask = jnp.full((16,), (1 << _SHIFT) - 1, jnp.int32)
    noff = jnp.full((16,), N, jnp.int32)

    hg = pltpu.async_copy(gall2_hbm.at[pair], g_v, sem_g)
    ebase = half * _EH
    h = pltpu.async_copy(ep_hbm.at[pl.ds(ebase, EC)], ep_b[0], sem[0])

    zero16 = jnp.zeros((16,), jnp.float32)

    def _init(i, c):
        acc_v[pl.ds(i * 16, 16)] = zero16
        return c
    lax.fori_loop(0, 2 * N // 16, _init, 0)
    hg.wait()

    for ci in range(_NCHH):
        buf = ep_b[ci % 2]
        hn = None
        if ci + 1 < _NCHH:
            nb = (ci + 1) % 2
            hn = pltpu.async_copy(ep_hbm.at[pl.ds(ebase + (ci + 1) * EC, EC)],
                                  ep_b[nb], sem[nb])
        h.wait()

        def _body(i, cc, buf=buf):
            e16s = [buf[pl.ds((i * _U + u) * 16, 16)] for u in range(_U)]
            svs = [e16 >> _SHIFT for e16 in e16s]
            dvs = [e16 & dmask for e16 in e16s]
            v0s = [plsc.load_gather(g_v, [s]) for s in svs]
            v1s = [plsc.load_gather(g_v, [s + noff]) for s in svs]
            for d, v0 in zip(dvs, v0s):
                plsc.addupdate_scatter(acc_v, [d], v0)
            for d, v1 in zip(dvs, v1s):
                plsc.addupdate_scatter(acc_v, [d + noff], v1)
            return cc
        lax.fori_loop(0, EC // 16 // _U, _body, 0)
        h = hn

    pltpu.sync_copy(acc_v, out_hbm.at[wid])


def _k2c_body(acca_ref, accb_ref, gall_ref, dinv_ref, bg_ref, o_ref):
    o_ref[...] = jnp.maximum(
        dinv_ref[...] * (acca_ref[...] + accb_ref[...] + gall_ref[...])
        + bg_ref[...], 0.0)


def _k2c(accA, accB, gall, dinv, bg):
    return pl.pallas_call(
        _k2c_body,
        out_shape=jax.ShapeDtypeStruct((D_H, N), jnp.float32),
    )(accA, accB, gall, dinv, bg.reshape(D_H, 1))


def _fc1_body(h_ref, w_ref, o_ref):
    i = pl.program_id(0)

    @pl.when(i == 0)
    def _():
        o_ref[...] = jnp.zeros_like(o_ref)

    o_ref[...] += jnp.dot(h_ref[...], w_ref[...],
                          preferred_element_type=jnp.float32)


def _k3(h2, W1):
    nblk = (N * D_H) // KBLK
    return pl.pallas_call(
        _fc1_body,
        grid=(nblk,),
        in_specs=[
            pl.BlockSpec((1, KBLK), lambda i: (0, i)),
            pl.BlockSpec((KBLK, 128), lambda i: (i, 0)),
        ],
        out_specs=pl.BlockSpec((1, 128), lambda i: (0, 0)),
        out_shape=jax.ShapeDtypeStruct((1, 128), jnp.float32),
    )(h2, W1)


def _fc2_body(h_ref, b1_ref, w2_ref, b2_ref, o_ref):
    h3 = jnp.maximum(h_ref[...] + b1_ref[...], 0.0)
    o_ref[...] = jnp.maximum(
        jnp.dot(h3, w2_ref[...], preferred_element_type=jnp.float32)
        + b2_ref[...], 0.0)


def _k4(h3pre, b1, W2, b2):
    return pl.pallas_call(
        _fc2_body,
        out_shape=jax.ShapeDtypeStruct((1, N), jnp.float32),
    )(h3pre, b1.reshape(1, 128), W2, b2.reshape(1, N))


def kernel(x, edge_index, Wg, bg, W1, b1, W2, b2):
    xwT, epacked = _k1(x, Wg, edge_index)
    ep = epacked.reshape(E)
    degP = _sc_deg(ep)
    dinv, gall = _kd(degP, xwT)
    acc2 = _sc_msg(ep, gall.reshape(16, 2 * N))
    accH = acc2.reshape(2, D_H, N)
    h1T = _k2c(accH[0], accH[1], gall, dinv, bg)
    h2 = h1T.T.reshape(1, N * D_H)
    h3pre = _k3(h2, W1)
    return _k4(h3pre, b1, W2, b2)


# speedup vs baseline: 1.0357x; 1.0357x over previous
import functools

import jax
import jax.numpy as jnp
from jax import lax
from jax.experimental import pallas as pl
from jax.experimental.pallas import tpu as pltpu
from jax.experimental.pallas import tpu_sc as plsc

N = 10000
E = 160000
D_IN = 128
D_H = 32
EC = 16000
KBLK = 16000
EPT = E // 32

_SHIFT = 14
_NCH = E // EC
_U = 8
assert E % EC == 0 and EC % (16 * _U) == 0 and N % 16 == 0
assert E % 32 == 0 and EPT % 8 == 0

_MESH = plsc.VectorSubcoreMesh(core_axis_name="c", subcore_axis_name="s")
_SC_PARAMS = pltpu.CompilerParams(needs_layout_passes=False)


def _k1_body(x_ref, wg_ref, e_ref, o_ref, p_ref):
    o_ref[...] = lax.dot_general(
        wg_ref[...], x_ref[...],
        dimension_numbers=(((0,), (1,)), ((), ())),
        preferred_element_type=jnp.float32)
    p_ref[...] = (e_ref[0:1, :] << _SHIFT) | e_ref[1:2, :]


def _k1(x, Wg, edge_index):
    return pl.pallas_call(
        _k1_body,
        out_shape=(
            jax.ShapeDtypeStruct((D_H, N), jnp.float32),
            jax.ShapeDtypeStruct((1, E), jnp.int32),
        ),
    )(x, Wg, edge_index)


@functools.partial(
    pl.kernel,
    mesh=_MESH,
    compiler_params=_SC_PARAMS,
    out_type=jax.ShapeDtypeStruct((32, N), jnp.float32),
    scratch_types=[
        pltpu.VMEM((EPT,), jnp.int32),
        pltpu.VMEM((N,), jnp.float32),
        pltpu.SemaphoreType.DMA,
    ],
)
def _sc_deg(ep_hbm, out_hbm, ep_v, deg_v, sem_e):
    wid = lax.axis_index("s") * 2 + lax.axis_index("c")

    h = pltpu.async_copy(ep_hbm.at[pl.ds(wid * EPT, EPT)], ep_v, sem_e)

    zero16 = jnp.zeros((16,), jnp.float32)

    def _init(i, c):
        deg_v[pl.ds(i * 16, 16)] = zero16
        return c
    lax.fori_loop(0, N // 16, _init, 0)
    h.wait()

    one16 = jnp.full((16,), 1.0, jnp.float32)
    dmask = jnp.full((16,), (1 << _SHIFT) - 1, jnp.int32)
    nfull = EPT // 16
    rem = EPT - nfull * 16

    def _body(i, c):
        e16 = ep_v[pl.ds(i * 16, 16)]
        plsc.addupdate_scatter(deg_v, [e16 & dmask], one16)
        return c
    lax.fori_loop(0, nfull, _body, 0)

    if rem:
        e16 = ep_v[pl.ds(EPT - 16, 16)]
        mask = lax.iota(jnp.int32, 16) >= (16 - rem)
        plsc.addupdate_scatter(deg_v, [e16 & dmask], one16, mask=mask)

    pltpu.sync_copy(deg_v, out_hbm.at[wid])


def _kd_body(degp_ref, xwt_ref, dinv_ref, gall_ref):
    deg = 1.0 + jnp.sum(degp_ref[...], axis=0, keepdims=True)
    dinv = lax.rsqrt(deg)
    dinv_ref[...] = dinv
    gall_ref[...] = dinv * xwt_ref[...]


def _kd(degP, xwT):
    return pl.pallas_call(
        _kd_body,
        out_shape=(
            jax.ShapeDtypeStruct((1, N), jnp.float32),
            jax.ShapeDtypeStruct((D_H, N), jnp.float32),
        ),
    )(degP, xwT)


@functools.partial(
    pl.kernel,
    mesh=_MESH,
    compiler_params=_SC_PARAMS,
    out_type=jax.ShapeDtypeStruct((D_H, N), jnp.float32),
    scratch_types=[
        pltpu.VMEM((EC,), jnp.int32),
        pltpu.VMEM((EC,), jnp.int32),
        pltpu.VMEM((N,), jnp.float32),
        pltpu.VMEM((N,), jnp.float32),
        pltpu.SemaphoreType.DMA,
        pltpu.SemaphoreType.DMA,
        pltpu.SemaphoreType.DMA,
    ],
)
def _sc_msg(ep_hbm, gall_hbm, out_hbm, ep_v0, ep_v1, g_v, acc_v,
            sem_0, sem_1, sem_g):
    wid = lax.axis_index("s") * 2 + lax.axis_index("c")
    ep_b = (ep_v0, ep_v1)
    sem = (sem_0, sem_1)
    dmask = jnp.full((16,), (1 << _SHIFT) - 1, jnp.int32)

    hg = pltpu.async_copy(gall_hbm.at[wid], g_v, sem_g)
    h = pltpu.async_copy(ep_hbm.at[pl.ds(0, EC)], ep_b[0], sem[0])

    zero16 = jnp.zeros((16,), jnp.float32)

    def _init(i, c):
        acc_v[pl.ds(i * 16, 16)] = zero16
        return c
    lax.fori_loop(0, N // 16, _init, 0)
    hg.wait()

    for ci in range(_NCH):
        buf = ep_b[ci % 2]
        hn = None
        if ci + 1 < _NCH:
            nb = (ci + 1) % 2
            hn = pltpu.async_copy(ep_hbm.at[pl.ds((ci + 1) * EC, EC)],
                                  ep_b[nb], sem[nb])
        h.wait()

        def _body(i, cc, buf=buf):
            e16s = [buf[pl.ds((i * _U + u) * 16, 16)] for u in range(_U)]
            vs = [plsc.load_gather(g_v, [e16 >> _SHIFT]) for e16 in e16s]
            for e16, v in zip(e16s, vs):
                plsc.addupdate_scatter(acc_v, [e16 & dmask], v)
            return cc
        lax.fori_loop(0, EC // 16 // _U, _body, 0)
        h = hn

    pltpu.sync_copy(acc_v, out_hbm.at[wid])


def _k2c_body(acc_ref, gall_ref, dinv_ref, bg_ref, o_ref):
    o_ref[...] = jnp.maximum(
        dinv_ref[...] * (acc_ref[...] + gall_ref[...]) + bg_ref[...], 0.0)


def _k2c(accT, gall, dinv, bg):
    return pl.pallas_call(
        _k2c_body,
        out_shape=jax.ShapeDtypeStruct((D_H, N), jnp.float32),
    )(accT, gall, dinv, bg.reshape(D_H, 1))


def _fc1_body(h_ref, w_ref, o_ref):
    i = pl.program_id(0)

    @pl.when(i == 0)
    def _():
        o_ref[...] = jnp.zeros_like(o_ref)

    o_ref[...] += jnp.dot(h_ref[...], w_ref[...],
                          preferred_element_type=jnp.float32)


def _k3(h2, W1):
    nblk = (N * D_H) // KBLK
    return pl.pallas_call(
        _fc1_body,
        grid=(nblk,),
        in_specs=[
            pl.BlockSpec((1, KBLK), lambda i: (0, i)),
            pl.BlockSpec((KBLK, 128), lambda i: (i, 0)),
        ],
        out_specs=pl.BlockSpec((1, 128), lambda i: (0, 0)),
        out_shape=jax.ShapeDtypeStruct((1, 128), jnp.float32),
    )(h2, W1)


def _fc2_body(h_ref, b1_ref, w2_ref, b2_ref, o_ref):
    h3 = jnp.maximum(h_ref[...] + b1_ref[...], 0.0)
    o_ref[...] = jnp.maximum(
        jnp.dot(h3, w2_ref[...], preferred_element_type=jnp.float32)
        + b2_ref[...], 0.0)


def _k4(h3pre, b1, W2, b2):
    return pl.pallas_call(
        _fc2_body,
        out_shape=jax.ShapeDtypeStruct((1, N), jnp.float32),
    )(h3pre, b1.reshape(1, 128), W2, b2.reshape(1, N))


def kernel(x, edge_index, Wg, bg, W1, b1, W2, b2):
    xwT, epacked = _k1(x, Wg, edge_index)
    ep = epacked.reshape(E)
    degP = _sc_deg(ep)
    dinv, gall = _kd(degP, xwT)
    accT = _sc_msg(ep, gall)
    h1T = _k2c(accT, gall, dinv, bg)
    h2 = h1T.T.reshape(1, N * D_H)
    h3pre = _k3(h2, W1)
    return _k4(h3pre, b1, W2, b2)
